# baseline (speedup 1.0000x reference)
import jax
import jax.numpy as jnp
from jax import lax
from jax.experimental import pallas as pl
from jax.experimental.pallas import tpu as pltpu
from jax.experimental.pallas import tpu_sc as plsc

N = 50000
HALF = 25000
D = 64
E = 800000
EPT = 51200
EP = EPT * 16
CHUNK = 2048
NCHUNK = EPT // CHUNK
BLK = 128
ACC = 25088
RPT = ACC // 16
NWB = 13

_mesh = plsc.VectorSubcoreMesh(
    core_axis_name="c", subcore_axis_name="s", num_cores=2, num_subcores=16)
_params = pltpu.CompilerParams(needs_layout_passes=False,
                               use_tc_tiling_on_sc=False)


def _wo_loc(s, j):
    return jnp.minimum(s * RPT + jnp.minimum(j * BLK, RPT - BLK), HALF - BLK)


def _deg_body(row_hbm, ones_hbm, zeros_hbm, deg_out,
              stage_r, sel0, sel1, onesb, zerosb, dacc, ssem0, ssem1):
    c = lax.axis_index("c")
    s = lax.axis_index("s")
    rbase = c * HALF
    iota = lax.iota(jnp.int32, 16)

    pltpu.sync_copy(ones_hbm, onesb)
    pltpu.sync_copy(zeros_hbm, zerosb)
    zb = s * RPT

    def z_body(j, carry):
        pltpu.sync_copy(zerosb, dacc.at[pl.ds(zb + j * BLK, BLK)])
        return carry

    lax.fori_loop(0, RPT // BLK, z_body, 0)
    pltpu.sync_copy(zerosb.at[pl.ds(0, RPT % BLK)],
                    dacc.at[pl.ds(zb + RPT - RPT % BLK, RPT % BLK)])
    plsc.subcore_barrier()

    e_base = s * EPT
    bpc = CHUNK // BLK

    def build(g, selbuf):
        for v in range(8):
            r = stage_r[pl.ds((g % bpc) * BLK + v * 16, 16)]
            owned = (r >= rbase) & (r < rbase + HALF)
            dmy = HALF + (v & 3) * 16 + iota
            sel = jnp.where(owned, r - rbase, dmy)
            selbuf[pl.ds(v * 16, 16)] = sel

    def g_body(g, carry):
        @pl.when(g % bpc == 0)
        def _():
            pltpu.sync_copy(
                row_hbm.at[pl.ds(e_base + (g // bpc) * CHUNK, CHUNK)],
                stage_r)

        @pl.when(g % 2 == 0)
        def _():
            @pl.when(g >= 2)
            def _():
                pltpu.make_async_copy(onesb, dacc.at[sel0], ssem0).wait()
            build(g, sel0)
            pltpu.async_copy(onesb, dacc.at[sel0], ssem0, add=True)

        @pl.when(g % 2 == 1)
        def _():
            @pl.when(g >= 2)
            def _():
                pltpu.make_async_copy(onesb, dacc.at[sel1], ssem1).wait()
            build(g, sel1)
            pltpu.async_copy(onesb, dacc.at[sel1], ssem1, add=True)

        return carry

    lax.fori_loop(0, NCHUNK * bpc, g_body, 0)
    pltpu.make_async_copy(onesb, dacc.at[sel0], ssem0).wait()
    pltpu.make_async_copy(onesb, dacc.at[sel1], ssem1).wait()
    plsc.subcore_barrier()

    def w_body(j, carry):
        loc = _wo_loc(s, j)
        pltpu.sync_copy(dacc.at[pl.ds(loc, BLK)], zerosb)
        pltpu.sync_copy(zerosb, deg_out.at[pl.ds(rbase + loc, BLK)])
        return carry

    lax.fori_loop(0, NWB, w_body, 0)


def _degree(row_p, ones16, zeros16):
    return pl.kernel(
        _deg_body,
        out_type=jax.ShapeDtypeStruct((N, 16), jnp.float32),
        mesh=_mesh,
        scratch_types=[
            pltpu.VMEM((CHUNK,), jnp.int32),
            pltpu.VMEM((BLK,), jnp.int32),
            pltpu.VMEM((BLK,), jnp.int32),
            pltpu.VMEM((BLK, 16), jnp.float32),
            pltpu.VMEM((BLK, 16), jnp.float32),
            pltpu.VMEM_SHARED((ACC, 16), jnp.float32),
            pltpu.SemaphoreType.DMA,
            pltpu.SemaphoreType.DMA,
        ],
        compiler_params=_params,
    )(row_p, ones16, zeros16)


def _make_round_body(final):
    def body(xs_hbm, row_hbm, col_hbm, dinv_hbm, aux_hbm, zeros_hbm,
             *refs):
        if final:
            (out_hbm, stage_r, stage_c, cr, cc, rb0, rb1, acc,
             gsem, ssem0, ssem1) = refs
        else:
            (hs_hbm, xh_hbm, stage_r, stage_c, cr, cc, rb0, rb1, acc,
             gsem, ssem0, ssem1) = refs
        c = lax.axis_index("c")
        s = lax.axis_index("s")
        cbase = c * HALF
        iota = lax.iota(jnp.int32, 16)

        pltpu.sync_copy(zeros_hbm, rb0)
        zb = s * RPT

        def z_body(j, carry):
            pltpu.sync_copy(rb0, acc.at[pl.ds(zb + j * BLK, BLK)])
            return carry

        lax.fori_loop(0, RPT // BLK, z_body, 0)
        pltpu.sync_copy(rb0.at[pl.ds(0, RPT % BLK)],
                        acc.at[pl.ds(zb + RPT - RPT % BLK, RPT % BLK)])
        plsc.subcore_barrier()

        e_base = s * EPT

        def chunk_body(k, carry):
            pltpu.sync_copy(row_hbm.at[pl.ds(e_base + k * CHUNK, CHUNK)],
                            stage_r)
            pltpu.sync_copy(col_hbm.at[pl.ds(e_base + k * CHUNK, CHUNK)],
                            stage_c)

            def comp_body(v, mv):
                r = stage_r[pl.ds(v * 16, 16)]
                cv = stage_c[pl.ds(v * 16, 16)]
                owned = (cv >= cbase) & (cv < cbase + HALF)
                inc = plsc.cumsum(owned.astype(jnp.int32))
                pos = mv + inc - 1
                hi = lax.shift_right_logical(pos, 7)
                lo = pos & (BLK - 1)
                plsc.store_scatter(cr, [hi, lo], r, mask=owned)
                plsc.store_scatter(cc, [hi, lo], cv - cbase, mask=owned)
                return mv + plsc.all_reduce_population_count(owned)

            mv = lax.fori_loop(0, CHUNK // 16, comp_body,
                               jnp.zeros((16,), jnp.int32))
            m = mv[0]
            nb = (m + BLK - 1) // BLK

            def pad_at(pos):
                hi = lax.shift_right_logical(pos, 7)
                lo = pos & (BLK - 1)
                plsc.store_scatter(cr, [hi, lo], iota)
                plsc.store_scatter(cc, [hi, lo], HALF + iota)

            pad_at(m + iota)

            def pad_body(j, carry):
                pad_at(j * 16 + iota)
                return carry

            lax.fori_loop(m // 16 + 1, nb * (BLK // 16), pad_body, 0)

            @pl.when(nb > 0)
            def _():
                pltpu.async_copy(xs_hbm.at[cr.at[0]], rb0, gsem)

            def blk_body(b, carry):
                nxt = b + 1

                @pl.when(b % 2 == 0)
                def _():
                    pltpu.make_async_copy(xs_hbm.at[cr.at[b]], rb0,
                                          gsem).wait()
                    pltpu.async_copy(rb0, acc.at[cc.at[b]], ssem0, add=True)

                    @pl.when(nxt < nb)
                    def _():
                        @pl.when(b >= 1)
                        def _():
                            pltpu.make_async_copy(
                                rb1, acc.at[cc.at[b - 1]], ssem1).wait()

                        pltpu.async_copy(xs_hbm.at[cr.at[nxt]], rb1, gsem)

                @pl.when(b % 2 == 1)
                def _():
                    pltpu.make_async_copy(xs_hbm.at[cr.at[b]], rb1,
                                          gsem).wait()
                    pltpu.async_copy(rb1, acc.at[cc.at[b]], ssem1, add=True)

                    @pl.when(nxt < nb)
                    def _():
                        pltpu.make_async_copy(
                            rb0, acc.at[cc.at[b - 1]], ssem0).wait()
                        pltpu.async_copy(xs_hbm.at[cr.at[nxt]], rb0, gsem)

                return carry

            lax.fori_loop(0, nb, blk_body, 0)

            @pl.when(nb >= 1)
            def _():
                last = nb - 1

                @pl.when(last % 2 == 0)
                def _():
                    pltpu.make_async_copy(rb0, acc.at[cc.at[last]],
                                          ssem0).wait()

                @pl.when(last % 2 == 1)
                def _():
                    pltpu.make_async_copy(rb1, acc.at[cc.at[last]],
                                          ssem1).wait()

            @pl.when(nb >= 2)
            def _():
                prev = nb - 2

                @pl.when(prev % 2 == 0)
                def _():
                    pltpu.make_async_copy(rb0, acc.at[cc.at[prev]],
                                          ssem0).wait()

                @pl.when(prev % 2 == 1)
                def _():
                    pltpu.make_async_copy(rb1, acc.at[cc.at[prev]],
                                          ssem1).wait()

            return carry

        lax.fori_loop(0, NCHUNK, chunk_body, 0)
        plsc.subcore_barrier()

        WB = BLK // 2

        def w_body(j, carry):
            loc = jnp.minimum(s * RPT + jnp.minimum(j * WB, RPT - WB),
                              HALF - WB)
            pltpu.sync_copy(acc.at[pl.ds(loc, WB)], rb0.at[pl.ds(0, WB)])
            pltpu.sync_copy(aux_hbm.at[pl.ds(cbase + loc, WB)],
                            rb0.at[pl.ds(WB, WB)])
            pltpu.sync_copy(dinv_hbm.at[pl.ds(cbase + loc, WB)],
                            rb1.at[pl.ds(0, WB)])

            def r_body(r, carry):
                for q in range(D // 16):
                    v = rb0[r, pl.ds(q * 16, 16)]
                    dv = rb1[r, pl.ds(q * 16, 16)]
                    a = rb0[WB + r, pl.ds(q * 16, 16)]
                    t = v * dv
                    rb1[WB + r, pl.ds(q * 16, 16)] = a + t
                    if not final:
                        rb0[r, pl.ds(q * 16, 16)] = t * dv
                return carry

            lax.fori_loop(0, WB, r_body, 0)
            if final:
                pltpu.sync_copy(rb1.at[pl.ds(WB, WB)],
                                out_hbm.at[pl.ds(cbase + loc, WB)])
            else:
                pltpu.sync_copy(rb0.at[pl.ds(0, WB)],
                                hs_hbm.at[pl.ds(cbase + loc, WB)])
                pltpu.sync_copy(rb1.at[pl.ds(WB, WB)],
                                xh_hbm.at[pl.ds(cbase + loc, WB)])
            return carry

        lax.fori_loop(0, RPT // WB + 1, w_body, 0)

    return body


_round1_body = _make_round_body(final=False)
_round2_body = _make_round_body(final=True)


def _round_scratch():
    return [
        pltpu.VMEM((CHUNK,), jnp.int32),
        pltpu.VMEM((CHUNK,), jnp.int32),
        pltpu.VMEM((CHUNK // BLK + 1, BLK), jnp.int32),
        pltpu.VMEM((CHUNK // BLK + 1, BLK), jnp.int32),
        pltpu.VMEM((BLK, D), jnp.float32),
        pltpu.VMEM((BLK, D), jnp.float32),
        pltpu.VMEM_SHARED((ACC, D), jnp.float32),
        pltpu.SemaphoreType.DMA,
        pltpu.SemaphoreType.DMA,
        pltpu.SemaphoreType.DMA,
    ]


def _gcn_round1(xs, row_p, col_p, dinvw, x, zeros64):
    return pl.kernel(
        _round1_body,
        out_type=[
            jax.ShapeDtypeStruct((N, D), jnp.float32),
            jax.ShapeDtypeStruct((N, D), jnp.float32),
        ],
        mesh=_mesh,
        scratch_types=_round_scratch(),
        compiler_params=_params,
    )(xs, row_p, col_p, dinvw, x, zeros64)


def _gcn_round2(hs, row_p, col_p, dinvw, xh, zeros64):
    return pl.kernel(
        _round2_body,
        out_type=jax.ShapeDtypeStruct((N, D), jnp.float32),
        mesh=_mesh,
        scratch_types=_round_scratch(),
        compiler_params=_params,
    )(hs, row_p, col_p, dinvw, xh, zeros64)


BK = 200
NBLK_HALF = HALF // BK


def _feat_body(pref, feat, degw, W1r, b1r, W2r, b2r, x_ref, xs_ref, dw_ref):
    i = pl.program_id(0)
    deg = degw[:, 0:1]
    dinv = jnp.where(deg > 0, lax.rsqrt(deg), 0.0)
    dw_ref[...] = jnp.broadcast_to(dinv, (BK, D))

    @pl.when(i < NBLK_HALF)
    def _():
        v = pref[...]
        nrm = jnp.maximum(jnp.sqrt(jnp.sum(v * v, axis=1, keepdims=True)),
                          1e-12)
        xb = v / nrm
        x_ref[...] = xb
        xs_ref[...] = xb * dinv

    @pl.when(i >= NBLK_HALF)
    def _():
        t = jnp.dot(feat[...], W1r[...],
                    preferred_element_type=jnp.float32) + b1r[...]
        t = jnp.where(t >= 0, t, 0.01 * t)
        t = jnp.dot(t, W2r[...],
                    preferred_element_type=jnp.float32) + b2r[...]
        nrm = jnp.maximum(jnp.sqrt(jnp.sum(t * t, axis=1, keepdims=True)),
                          1e-12)
        xb = t / nrm
        x_ref[...] = xb
        xs_ref[...] = xb * dinv


def _feat(features, preference, deg_wide, W1, b1, W2, b2):
    nb = 2 * NBLK_HALF
    return pl.pallas_call(
        _feat_body,
        out_shape=[
            jax.ShapeDtypeStruct((N, D), jnp.float32),
            jax.ShapeDtypeStruct((N, D), jnp.float32),
            jax.ShapeDtypeStruct((N, D), jnp.float32),
        ],
        grid=(nb,),
        in_specs=[
            pl.BlockSpec((BK, D), lambda i: (jnp.minimum(i, NBLK_HALF - 1), 0)),
            pl.BlockSpec((BK, 128),
                         lambda i: (jnp.maximum(i - NBLK_HALF, 0), 0)),
            pl.BlockSpec((BK, 16), lambda i: (i, 0)),
            pl.BlockSpec((128, 256), lambda i: (0, 0)),
            pl.BlockSpec((256,), lambda i: (0,)),
            pl.BlockSpec((256, D), lambda i: (0, 0)),
            pl.BlockSpec((D,), lambda i: (0,)),
        ],
        out_specs=[
            pl.BlockSpec((BK, D), lambda i: (i, 0)),
            pl.BlockSpec((BK, D), lambda i: (i, 0)),
            pl.BlockSpec((BK, D), lambda i: (i, 0)),
        ],
    )(preference, features, deg_wide, W1, b1, W2, b2)


def kernel(edge_index, features, preference, W1, b1, W2, b2):
    row = edge_index[0].astype(jnp.int32)
    col = edge_index[1].astype(jnp.int32)
    pad = jnp.full((EP - E,), N, jnp.int32)
    row_p = jnp.concatenate([row, pad])
    col_p = jnp.concatenate([col, pad])
    ones16 = jnp.ones((BLK, 16), jnp.float32)
    zeros16 = jnp.zeros((BLK, 16), jnp.float32)
    zeros64 = jnp.zeros((BLK, D), jnp.float32)

    deg_wide = _degree(row_p, ones16, zeros16)
    x, xs, dinvw = _feat(features, preference, deg_wide, W1, b1, W2, b2)
    hs, xh = _gcn_round1(xs, row_p, col_p, dinvw, x, zeros64)
    return _gcn_round2(hs, row_p, col_p, dinvw, xh, zeros64)


# speedup vs baseline: 1.0272x; 1.0272x over previous
import jax
import jax.numpy as jnp
from jax import lax
from jax.experimental import pallas as pl
from jax.experimental.pallas import tpu as pltpu
from jax.experimental.pallas import tpu_sc as plsc

N = 50000
HALF = 25000
D = 64
E = 800000
EPT = 51200
EP = EPT * 16
CHUNK = 2048
NCHUNK = EPT // CHUNK
BLK = 128
ACC = 25088
RPT = ACC // 16
NWB = 13

_mesh = plsc.VectorSubcoreMesh(
    core_axis_name="c", subcore_axis_name="s", num_cores=2, num_subcores=16)
_params = pltpu.CompilerParams(needs_layout_passes=False,
                               use_tc_tiling_on_sc=False)


def _wo_loc(s, j):
    return jnp.minimum(s * RPT + jnp.minimum(j * BLK, RPT - BLK), HALF - BLK)


def _deg_body(row_hbm, ones_hbm, zeros_hbm, deg_out,
              stage_r, sel0, sel1, onesb, zerosb, dacc, ssem0, ssem1):
    c = lax.axis_index("c")
    s = lax.axis_index("s")
    rbase = c * HALF
    iota = lax.iota(jnp.int32, 16)

    pltpu.sync_copy(ones_hbm, onesb)
    pltpu.sync_copy(zeros_hbm, zerosb)
    zb = s * RPT

    def z_body(j, carry):
        pltpu.sync_copy(zerosb, dacc.at[pl.ds(zb + j * BLK, BLK)])
        return carry

    lax.fori_loop(0, RPT // BLK, z_body, 0)
    pltpu.sync_copy(zerosb.at[pl.ds(0, RPT % BLK)],
                    dacc.at[pl.ds(zb + RPT - RPT % BLK, RPT % BLK)])
    plsc.subcore_barrier()

    e_base = s * EPT
    bpc = CHUNK // BLK

    def build(g, selbuf):
        for v in range(8):
            r = stage_r[pl.ds((g % bpc) * BLK + v * 16, 16)]
            owned = (r >= rbase) & (r < rbase + HALF)
            dmy = HALF + (v & 3) * 16 + iota
            sel = jnp.where(owned, r - rbase, dmy)
            selbuf[pl.ds(v * 16, 16)] = sel

    def g_body(g, carry):
        @pl.when(g % bpc == 0)
        def _():
            pltpu.sync_copy(
                row_hbm.at[pl.ds(e_base + (g // bpc) * CHUNK, CHUNK)],
                stage_r)

        @pl.when(g % 2 == 0)
        def _():
            @pl.when(g >= 2)
            def _():
                pltpu.make_async_copy(onesb, dacc.at[sel0], ssem0).wait()
            build(g, sel0)
            pltpu.async_copy(onesb, dacc.at[sel0], ssem0, add=True)

        @pl.when(g % 2 == 1)
        def _():
            @pl.when(g >= 2)
            def _():
                pltpu.make_async_copy(onesb, dacc.at[sel1], ssem1).wait()
            build(g, sel1)
            pltpu.async_copy(onesb, dacc.at[sel1], ssem1, add=True)

        return carry

    lax.fori_loop(0, NCHUNK * bpc, g_body, 0)
    pltpu.make_async_copy(onesb, dacc.at[sel0], ssem0).wait()
    pltpu.make_async_copy(onesb, dacc.at[sel1], ssem1).wait()
    plsc.subcore_barrier()

    def w_body(j, carry):
        loc = _wo_loc(s, j)
        pltpu.sync_copy(dacc.at[pl.ds(loc, BLK)], zerosb)
        pltpu.sync_copy(zerosb, deg_out.at[pl.ds(rbase + loc, BLK)])
        return carry

    lax.fori_loop(0, NWB, w_body, 0)


def _degree(row_p, ones16, zeros16):
    return pl.kernel(
        _deg_body,
        out_type=jax.ShapeDtypeStruct((N, 16), jnp.float32),
        mesh=_mesh,
        scratch_types=[
            pltpu.VMEM((CHUNK,), jnp.int32),
            pltpu.VMEM((BLK,), jnp.int32),
            pltpu.VMEM((BLK,), jnp.int32),
            pltpu.VMEM((BLK, 16), jnp.float32),
            pltpu.VMEM((BLK, 16), jnp.float32),
            pltpu.VMEM_SHARED((ACC, 16), jnp.float32),
            pltpu.SemaphoreType.DMA,
            pltpu.SemaphoreType.DMA,
        ],
        compiler_params=_params,
    )(row_p, ones16, zeros16)


def _make_round_body(final):
    def body(xs_hbm, row_hbm, col_hbm, dinv_hbm, aux_hbm, zeros_hbm,
             *refs):
        if final:
            (out_hbm, stage_r, stage_c, cr, cc, rb0, rb1, acc,
             gsem, ssem0, ssem1) = refs
        else:
            (hs_hbm, xh_hbm, stage_r, stage_c, cr, cc, rb0, rb1, acc,
             gsem, ssem0, ssem1) = refs
        c = lax.axis_index("c")
        s = lax.axis_index("s")
        cbase = c * HALF
        iota = lax.iota(jnp.int32, 16)

        pltpu.sync_copy(zeros_hbm, rb0)
        zb = s * RPT

        def z_body(j, carry):
            pltpu.sync_copy(rb0, acc.at[pl.ds(zb + j * BLK, BLK)])
            return carry

        lax.fori_loop(0, RPT // BLK, z_body, 0)
        pltpu.sync_copy(rb0.at[pl.ds(0, RPT % BLK)],
                        acc.at[pl.ds(zb + RPT - RPT % BLK, RPT % BLK)])
        plsc.subcore_barrier()

        e_base = s * EPT

        def chunk_body(k, carry):
            pltpu.sync_copy(row_hbm.at[pl.ds(e_base + k * CHUNK, CHUNK)],
                            stage_r)
            pltpu.sync_copy(col_hbm.at[pl.ds(e_base + k * CHUNK, CHUNK)],
                            stage_c)

            def comp_body(v, mv):
                r = stage_r[pl.ds(v * 16, 16)]
                cv = stage_c[pl.ds(v * 16, 16)]
                owned = (cv >= cbase) & (cv < cbase + HALF)
                inc = plsc.cumsum(owned.astype(jnp.int32))
                pos = mv + inc - 1
                hi = lax.shift_right_logical(pos, 7)
                lo = pos & (BLK - 1)
                plsc.store_scatter(cr, [hi, lo], r, mask=owned)
                plsc.store_scatter(cc, [hi, lo], cv - cbase, mask=owned)
                return mv + plsc.all_reduce_population_count(owned)

            mv = lax.fori_loop(0, CHUNK // 16, comp_body,
                               jnp.zeros((16,), jnp.int32))
            m = mv[0]
            nb = (m + BLK - 1) // BLK

            def pad_at(pos):
                hi = lax.shift_right_logical(pos, 7)
                lo = pos & (BLK - 1)
                sr = ((pos + s * 37 + k) * 797) % N
                plsc.store_scatter(cr, [hi, lo], sr)
                plsc.store_scatter(cc, [hi, lo], HALF + iota)

            pad_at(m + iota)

            def pad_body(j, carry):
                pad_at(j * 16 + iota)
                return carry

            lax.fori_loop(m // 16 + 1, nb * (BLK // 16), pad_body, 0)

            @pl.when(nb > 0)
            def _():
                pltpu.async_copy(xs_hbm.at[cr.at[0]], rb0, gsem)

            def blk_body(b, carry):
                nxt = b + 1

                @pl.when(b % 2 == 0)
                def _():
                    pltpu.make_async_copy(xs_hbm.at[cr.at[b]], rb0,
                                          gsem).wait()
                    pltpu.async_copy(rb0, acc.at[cc.at[b]], ssem0, add=True)

                    @pl.when(nxt < nb)
                    def _():
                        @pl.when(b >= 1)
                        def _():
                            pltpu.make_async_copy(
                                rb1, acc.at[cc.at[b - 1]], ssem1).wait()

                        pltpu.async_copy(xs_hbm.at[cr.at[nxt]], rb1, gsem)

                @pl.when(b % 2 == 1)
                def _():
                    pltpu.make_async_copy(xs_hbm.at[cr.at[b]], rb1,
                                          gsem).wait()
                    pltpu.async_copy(rb1, acc.at[cc.at[b]], ssem1, add=True)

                    @pl.when(nxt < nb)
                    def _():
                        pltpu.make_async_copy(
                            rb0, acc.at[cc.at[b - 1]], ssem0).wait()
                        pltpu.async_copy(xs_hbm.at[cr.at[nxt]], rb0, gsem)

                return carry

            lax.fori_loop(0, nb, blk_body, 0)

            @pl.when(nb >= 1)
            def _():
                last = nb - 1

                @pl.when(last % 2 == 0)
                def _():
                    pltpu.make_async_copy(rb0, acc.at[cc.at[last]],
                                          ssem0).wait()

                @pl.when(last % 2 == 1)
                def _():
                    pltpu.make_async_copy(rb1, acc.at[cc.at[last]],
                                          ssem1).wait()

            @pl.when(nb >= 2)
            def _():
                prev = nb - 2

                @pl.when(prev % 2 == 0)
                def _():
                    pltpu.make_async_copy(rb0, acc.at[cc.at[prev]],
                                          ssem0).wait()

                @pl.when(prev % 2 == 1)
                def _():
                    pltpu.make_async_copy(rb1, acc.at[cc.at[prev]],
                                          ssem1).wait()

            return carry

        lax.fori_loop(0, NCHUNK, chunk_body, 0)
        plsc.subcore_barrier()

        WB = BLK // 2

        def w_body(j, carry):
            loc = jnp.minimum(s * RPT + jnp.minimum(j * WB, RPT - WB),
                              HALF - WB)
            pltpu.sync_copy(acc.at[pl.ds(loc, WB)], rb0.at[pl.ds(0, WB)])
            pltpu.sync_copy(aux_hbm.at[pl.ds(cbase + loc, WB)],
                            rb0.at[pl.ds(WB, WB)])
            pltpu.sync_copy(dinv_hbm.at[pl.ds(cbase + loc, WB)],
                            rb1.at[pl.ds(0, WB)])

            def r_body(r, carry):
                for q in range(D // 16):
                    v = rb0[r, pl.ds(q * 16, 16)]
                    dv = rb1[r, pl.ds(q * 16, 16)]
                    a = rb0[WB + r, pl.ds(q * 16, 16)]
                    t = v * dv
                    rb1[WB + r, pl.ds(q * 16, 16)] = a + t
                    if not final:
                        rb0[r, pl.ds(q * 16, 16)] = t * dv
                return carry

            lax.fori_loop(0, WB, r_body, 0)
            if final:
                pltpu.sync_copy(rb1.at[pl.ds(WB, WB)],
                                out_hbm.at[pl.ds(cbase + loc, WB)])
            else:
                pltpu.sync_copy(rb0.at[pl.ds(0, WB)],
                                hs_hbm.at[pl.ds(cbase + loc, WB)])
                pltpu.sync_copy(rb1.at[pl.ds(WB, WB)],
                                xh_hbm.at[pl.ds(cbase + loc, WB)])
            return carry

        lax.fori_loop(0, RPT // WB + 1, w_body, 0)

    return body


_round1_body = _make_round_body(final=False)
_round2_body = _make_round_body(final=True)


def _round_scratch():
    return [
        pltpu.VMEM((CHUNK,), jnp.int32),
        pltpu.VMEM((CHUNK,), jnp.int32),
        pltpu.VMEM((CHUNK // BLK + 1, BLK), jnp.int32),
        pltpu.VMEM((CHUNK // BLK + 1, BLK), jnp.int32),
        pltpu.VMEM((BLK, D), jnp.float32),
        pltpu.VMEM((BLK, D), jnp.float32),
        pltpu.VMEM_SHARED((ACC, D), jnp.float32),
        pltpu.SemaphoreType.DMA,
        pltpu.SemaphoreType.DMA,
        pltpu.SemaphoreType.DMA,
    ]


def _gcn_round1(xs, row_p, col_p, dinvw, x, zeros64):
    return pl.kernel(
        _round1_body,
        out_type=[
            jax.ShapeDtypeStruct((N, D), jnp.float32),
            jax.ShapeDtypeStruct((N, D), jnp.float32),
        ],
        mesh=_mesh,
        scratch_types=_round_scratch(),
        compiler_params=_params,
    )(xs, row_p, col_p, dinvw, x, zeros64)


def _gcn_round2(hs, row_p, col_p, dinvw, xh, zeros64):
    return pl.kernel(
        _round2_body,
        out_type=jax.ShapeDtypeStruct((N, D), jnp.float32),
        mesh=_mesh,
        scratch_types=_round_scratch(),
        compiler_params=_params,
    )(hs, row_p, col_p, dinvw, xh, zeros64)


BK = 200
NBLK_HALF = HALF // BK


def _feat_body(pref, feat, degw, W1r, b1r, W2r, b2r, x_ref, xs_ref, dw_ref):
    i = pl.program_id(0)
    deg = degw[:, 0:1]
    dinv = jnp.where(deg > 0, lax.rsqrt(deg), 0.0)
    dw_ref[...] = jnp.broadcast_to(dinv, (BK, D))

    @pl.when(i < NBLK_HALF)
    def _():
        v = pref[...]
        nrm = jnp.maximum(jnp.sqrt(jnp.sum(v * v, axis=1, keepdims=True)),
                          1e-12)
        xb = v / nrm
        x_ref[...] = xb
        xs_ref[...] = xb * dinv

    @pl.when(i >= NBLK_HALF)
    def _():
        t = jnp.dot(feat[...], W1r[...],
                    preferred_element_type=jnp.float32) + b1r[...]
        t = jnp.where(t >= 0, t, 0.01 * t)
        t = jnp.dot(t, W2r[...],
                    preferred_element_type=jnp.float32) + b2r[...]
        nrm = jnp.maximum(jnp.sqrt(jnp.sum(t * t, axis=1, keepdims=True)),
                          1e-12)
        xb = t / nrm
        x_ref[...] = xb
        xs_ref[...] = xb * dinv


def _feat(features, preference, deg_wide, W1, b1, W2, b2):
    nb = 2 * NBLK_HALF
    return pl.pallas_call(
        _feat_body,
        out_shape=[
            jax.ShapeDtypeStruct((N, D), jnp.float32),
            jax.ShapeDtypeStruct((N, D), jnp.float32),
            jax.ShapeDtypeStruct((N, D), jnp.float32),
        ],
        grid=(nb,),
        in_specs=[
            pl.BlockSpec((BK, D), lambda i: (jnp.minimum(i, NBLK_HALF - 1), 0)),
            pl.BlockSpec((BK, 128),
                         lambda i: (jnp.maximum(i - NBLK_HALF, 0), 0)),
            pl.BlockSpec((BK, 16), lambda i: (i, 0)),
            pl.BlockSpec((128, 256), lambda i: (0, 0)),
            pl.BlockSpec((256,), lambda i: (0,)),
            pl.BlockSpec((256, D), lambda i: (0, 0)),
            pl.BlockSpec((D,), lambda i: (0,)),
        ],
        out_specs=[
            pl.BlockSpec((BK, D), lambda i: (i, 0)),
            pl.BlockSpec((BK, D), lambda i: (i, 0)),
            pl.BlockSpec((BK, D), lambda i: (i, 0)),
        ],
    )(preference, features, deg_wide, W1, b1, W2, b2)


def kernel(edge_index, features, preference, W1, b1, W2, b2):
    row = edge_index[0].astype(jnp.int32)
    col = edge_index[1].astype(jnp.int32)
    pad = jnp.full((EP - E,), N, jnp.int32)
    row_p = jnp.concatenate([row, pad])
    col_p = jnp.concatenate([col, pad])
    ones16 = jnp.ones((BLK, 16), jnp.float32)
    zeros16 = jnp.zeros((BLK, 16), jnp.float32)
    zeros64 = jnp.zeros((BLK, D), jnp.float32)

    deg_wide = _degree(row_p, ones16, zeros16)
    x, xs, dinvw = _feat(features, preference, deg_wide, W1, b1, W2, b2)
    hs, xh = _gcn_round1(xs, row_p, col_p, dinvw, x, zeros64)
    return _gcn_round2(hs, row_p, col_p, dinvw, xh, zeros64)
